# baseline (speedup 1.0000x reference)
import functools
import jax
import jax.numpy as jnp
from jax import lax
from jax.experimental import pallas as pl
from jax.experimental.pallas import tpu as pltpu

K = 64
ENC_T = 512
ENC_F = 1024
DEC_T = 128


def _encode_body(x_ref, w_ref, benc_ref, bdec_ref, out_ref):
    xm = x_ref[...] - bdec_ref[...]
    pre = lax.dot_general(
        xm, w_ref[...],
        dimension_numbers=(((1,), (1,)), ((), ())),
        preferred_element_type=jnp.float32,
    )
    out_ref[...] = pre + benc_ref[...]


def _key_to_f32(kk):
    return lax.bitcast_convert_type(
        jnp.where(kk >= 0, kk, kk ^ jnp.int32(0x7FFFFFFF)), jnp.float32)


def _row_kth_threshold_f(pre, k, ones):
    n_rows = pre.shape[0]

    def cnt(candf):
        ind = jnp.where(pre >= candf, 1.0, 0.0)
        return lax.dot_general(
            ind, ones,
            dimension_numbers=(((1,), (0,)), ((), ())),
            preferred_element_type=jnp.float32,
        )[:, :1]

    imin = jnp.int32(-0x80000000)
    t = jnp.full((n_rows, 1), imin, dtype=jnp.int32)
    c = cnt(jnp.float32(0.0))
    t = jnp.where(c >= k, jnp.int32(0), t)
    for b in range(30, -1, -1):
        cand = t + jnp.int32(1 << b)
        c = cnt(_key_to_f32(cand))
        t = jnp.where(c >= k, cand, t)
    return _key_to_f32(t)


def _fused_body(pre_ref, w_ref, bdec_ref, ones_ref, out_ref, acts0, acts1):
    i = pl.program_id(0)
    n = pl.num_programs(0)

    @pl.when(i < n - 1)
    def _search():
        pre = pre_ref[...]
        tf = _row_kth_threshold_f(pre, K, ones_ref[...])
        acts = jnp.where(pre >= tf, pre, 0.0).astype(jnp.bfloat16)

        @pl.when(lax.rem(i, 2) == 0)
        def _():
            acts0[...] = acts

        @pl.when(lax.rem(i, 2) == 1)
        def _():
            acts1[...] = acts

    @pl.when(i > 0)
    def _decode():
        @pl.when(lax.rem(i, 2) == 1)
        def _():
            out_ref[...] = lax.dot_general(
                acts0[...], w_ref[...],
                dimension_numbers=(((1,), (0,)), ((), ())),
                preferred_element_type=jnp.float32,
            ) + bdec_ref[...]

        @pl.when(lax.rem(i, 2) == 0)
        def _():
            out_ref[...] = lax.dot_general(
                acts1[...], w_ref[...],
                dimension_numbers=(((1,), (0,)), ((), ())),
                preferred_element_type=jnp.float32,
            ) + bdec_ref[...]


@jax.jit
def kernel(x, W_enc, b_enc, W_dec, b_dec):
    B, S, D = x.shape
    N = B * S
    F = W_enc.shape[0]
    x2 = x.reshape(N, D)

    pre = pl.pallas_call(
        _encode_body,
        grid=(F // ENC_F, N // ENC_T),
        in_specs=[
            pl.BlockSpec((ENC_T, D), lambda f, i: (i, 0)),
            pl.BlockSpec((ENC_F, D), lambda f, i: (f, 0)),
            pl.BlockSpec((1, ENC_F), lambda f, i: (0, f)),
            pl.BlockSpec((1, D), lambda f, i: (0, 0)),
        ],
        out_specs=pl.BlockSpec((ENC_T, ENC_F), lambda f, i: (i, f)),
        out_shape=jax.ShapeDtypeStruct((N, F), jnp.float32),
    )(x2, W_enc, b_enc.reshape(1, F), b_dec.reshape(1, D))

    nblk = N // DEC_T
    rec = pl.pallas_call(
        _fused_body,
        grid=(nblk + 1,),
        in_specs=[
            pl.BlockSpec((DEC_T, F), lambda i: (jnp.minimum(i, nblk - 1), 0)),
            pl.BlockSpec((F, D), lambda i: (0, 0)),
            pl.BlockSpec((1, D), lambda i: (0, 0)),
            pl.BlockSpec((F, 128), lambda i: (0, 0)),
        ],
        out_specs=pl.BlockSpec((DEC_T, D), lambda i: (jnp.maximum(i - 1, 0), 0)),
        out_shape=jax.ShapeDtypeStruct((N, D), jnp.float32),
        scratch_shapes=[
            pltpu.VMEM((DEC_T, F), jnp.bfloat16),
            pltpu.VMEM((DEC_T, F), jnp.bfloat16),
        ],
    )(pre, W_dec.astype(jnp.bfloat16), b_dec.reshape(1, D),
      jnp.ones((F, 128), jnp.float32))

    return rec.reshape(B, S, D)


# speedup vs baseline: 1.3473x; 1.3473x over previous
import functools
import jax
import jax.numpy as jnp
from jax import lax
from jax.experimental import pallas as pl
from jax.experimental.pallas import tpu as pltpu

K = 64
ENC_T = 512
ENC_F = 1024
DEC_T = 128


def _encode_body(x_ref, w_ref, benc_ref, bdec_ref, out_ref):
    xm = x_ref[...] - bdec_ref[...]
    pre = lax.dot_general(
        xm, w_ref[...],
        dimension_numbers=(((1,), (1,)), ((), ())),
        preferred_element_type=jnp.float32,
    )
    out_ref[...] = pre + benc_ref[...]


def _key_to_f32(kk):
    return lax.bitcast_convert_type(
        jnp.where(kk >= 0, kk, kk ^ jnp.int32(0x7FFFFFFF)), jnp.float32)


def _row_kth_threshold_f(pre, k):
    n_rows = pre.shape[0]

    def cnt(candf):
        return jnp.sum(jnp.where(pre >= candf, 1.0, 0.0),
                       axis=1, keepdims=True)

    imin = jnp.int32(-0x80000000)
    t = jnp.full((n_rows, 1), imin, dtype=jnp.int32)
    c = cnt(jnp.float32(0.0))
    t = jnp.where(c >= k, jnp.int32(0), t)
    for b in range(30, -1, -1):
        cand = t + jnp.int32(1 << b)
        c = cnt(_key_to_f32(cand))
        t = jnp.where(c >= k, cand, t)
    return _key_to_f32(t)


def _fused_body(pre_ref, w_ref, bdec_ref, out_ref, acts0, acts1):
    i = pl.program_id(0)
    n = pl.num_programs(0)

    @pl.when(i < n - 1)
    def _search():
        pre = pre_ref[...]
        tf = _row_kth_threshold_f(pre, K)
        acts = jnp.where(pre >= tf, pre, 0.0).astype(jnp.bfloat16)

        @pl.when(lax.rem(i, 2) == 0)
        def _():
            acts0[...] = acts

        @pl.when(lax.rem(i, 2) == 1)
        def _():
            acts1[...] = acts

    @pl.when(i > 0)
    def _decode():
        @pl.when(lax.rem(i, 2) == 1)
        def _():
            out_ref[...] = lax.dot_general(
                acts0[...], w_ref[...],
                dimension_numbers=(((1,), (0,)), ((), ())),
                preferred_element_type=jnp.float32,
            ) + bdec_ref[...]

        @pl.when(lax.rem(i, 2) == 0)
        def _():
            out_ref[...] = lax.dot_general(
                acts1[...], w_ref[...],
                dimension_numbers=(((1,), (0,)), ((), ())),
                preferred_element_type=jnp.float32,
            ) + bdec_ref[...]


@jax.jit
def kernel(x, W_enc, b_enc, W_dec, b_dec):
    B, S, D = x.shape
    N = B * S
    F = W_enc.shape[0]
    x2 = x.reshape(N, D)

    pre = pl.pallas_call(
        _encode_body,
        grid=(F // ENC_F, N // ENC_T),
        in_specs=[
            pl.BlockSpec((ENC_T, D), lambda f, i: (i, 0)),
            pl.BlockSpec((ENC_F, D), lambda f, i: (f, 0)),
            pl.BlockSpec((1, ENC_F), lambda f, i: (0, f)),
            pl.BlockSpec((1, D), lambda f, i: (0, 0)),
        ],
        out_specs=pl.BlockSpec((ENC_T, ENC_F), lambda f, i: (i, f)),
        out_shape=jax.ShapeDtypeStruct((N, F), jnp.float32),
    )(x2, W_enc, b_enc.reshape(1, F), b_dec.reshape(1, D))

    nblk = N // DEC_T
    rec = pl.pallas_call(
        _fused_body,
        grid=(nblk + 1,),
        in_specs=[
            pl.BlockSpec((DEC_T, F), lambda i: (jnp.minimum(i, nblk - 1), 0)),
            pl.BlockSpec((F, D), lambda i: (0, 0)),
            pl.BlockSpec((1, D), lambda i: (0, 0)),
        ],
        out_specs=pl.BlockSpec((DEC_T, D), lambda i: (jnp.maximum(i - 1, 0), 0)),
        out_shape=jax.ShapeDtypeStruct((N, D), jnp.float32),
        scratch_shapes=[
            pltpu.VMEM((DEC_T, F), jnp.bfloat16),
            pltpu.VMEM((DEC_T, F), jnp.bfloat16),
        ],
    )(pre, W_dec.astype(jnp.bfloat16), b_dec.reshape(1, D))

    return rec.reshape(B, S, D)
